# Initial kernel scaffold; baseline (speedup 1.0000x reference)
import jax
import jax.numpy as jnp
from jax.experimental import pallas as pl


def kernel(x, edge_index, edge_weight, W, b):
    raise NotImplementedError("write your pallas kernel here")


# speedup vs baseline: 12.7534x; 12.7534x over previous
import functools

import jax
import jax.numpy as jnp
from jax import lax
from jax.experimental import pallas as pl
from jax.experimental.pallas import tpu as pltpu
from jax.experimental.pallas import tpu_sc as plsc

NC = 2
NS = 16
NW = NC * NS
LANES = 16
CHUNK = 128

_MESH = plsc.VectorSubcoreMesh(core_axis_name="c", subcore_axis_name="s")

_GATHER_DNUMS = lax.GatherDimensionNumbers(
    offset_dims=(), collapsed_slice_dims=(0,), start_index_map=(0,))


def _bcast_lane(vec16, r):
    idx = jnp.full((LANES, 1), r, jnp.int32)
    return lax.gather(vec16, idx, _GATHER_DNUMS, (1,),
                      mode=lax.GatherScatterMode.PROMISE_IN_BOUNDS)


DEGW = 128


def _sc_degree(col, ew, n):
    e = col.shape[0]
    assert e % CHUNK == 0
    nchunk = e // CHUNK
    zrows = 80
    assert n % zrows == 0
    nzch = n // zrows

    @functools.partial(
        pl.kernel,
        out_type=jax.ShapeDtypeStruct((NC, n, DEGW), jnp.float32),
        mesh=_MESH,
        scratch_types=[
            pltpu.VMEM((CHUNK,), jnp.int32),
            pltpu.VMEM((CHUNK,), jnp.float32),
            pltpu.VMEM((CHUNK, DEGW), jnp.float32),
            pltpu.VMEM((zrows, DEGW), jnp.float32),
            pltpu.VMEM((LANES,), jnp.float32),
            pltpu.VMEM_SHARED((n, DEGW), jnp.float32),
        ],
    )
    def deg_kernel(col_hbm, ew_hbm, out_hbm, col_v, ew_v, rowbuf, zbuf,
                   ones_v, acc):
        c = lax.axis_index("c")
        s = lax.axis_index("s")
        w = s * NC + c
        ones_v[...] = jnp.ones((LANES,), jnp.float32)

        @pl.loop(0, zrows)
        def _(r):
            @pl.loop(0, DEGW // LANES)
            def _(k):
                zbuf[r, pl.ds(k * LANES, LANES)] = jnp.zeros((LANES,),
                                                             jnp.float32)

        @pl.loop(s, nzch, step=NS)
        def _(rc):
            pltpu.sync_copy(zbuf, acc.at[pl.ds(rc * zrows, zrows)])

        plsc.subcore_barrier()

        @pl.loop(w, nchunk, step=NW)
        def _(ci):
            base = ci * CHUNK
            pltpu.sync_copy(col_hbm.at[pl.ds(base, CHUNK)], col_v)
            pltpu.sync_copy(ew_hbm.at[pl.ds(base, CHUNK)], ew_v)

            @pl.loop(0, CHUNK // LANES)
            def _(q):
                ew16 = ew_v[pl.ds(q * LANES, LANES)]
                for r in range(LANES):
                    bvec = ones_v[...] * _bcast_lane(ew16, r)
                    for k in range(DEGW // LANES):
                        rowbuf[q * LANES + r, pl.ds(k * LANES, LANES)] = bvec

            pltpu.sync_copy(rowbuf, acc.at[col_v], add=True)

        plsc.subcore_barrier()

        @pl.loop(s, nzch, step=NS)
        def _(rc):
            pltpu.sync_copy(acc.at[pl.ds(rc * zrows, zrows)],
                            out_hbm.at[c, pl.ds(rc * zrows, zrows)])

    return deg_kernel(col, ew)


def _sc_aggregate(g, row, col, ew, n):
    e = row.shape[0]
    d = g.shape[1]
    assert e % CHUNK == 0 and d % LANES == 0
    nchunk = e // CHUNK
    zrows = 80
    assert n % zrows == 0
    nzch = n // zrows

    @functools.partial(
        pl.kernel,
        out_type=jax.ShapeDtypeStruct((NC, n, d), jnp.float32),
        mesh=_MESH,
        scratch_types=[
            pltpu.VMEM((CHUNK,), jnp.int32),
            pltpu.VMEM((CHUNK,), jnp.int32),
            pltpu.VMEM((CHUNK,), jnp.float32),
            pltpu.VMEM((CHUNK, d), jnp.float32),
            pltpu.VMEM((zrows, d), jnp.float32),
            pltpu.VMEM_SHARED((n, d), jnp.float32),
        ],
    )
    def agg_kernel(g_hbm, row_hbm, col_hbm, ew_hbm, out_hbm,
                   row_v, col_v, ew_v, msg, zbuf, acc):
        c = lax.axis_index("c")
        s = lax.axis_index("s")
        w = s * NC + c

        @pl.loop(0, zrows)
        def _(r):
            @pl.loop(0, d // LANES)
            def _(k):
                zbuf[r, pl.ds(k * LANES, LANES)] = jnp.zeros((LANES,),
                                                             jnp.float32)

        @pl.loop(s, nzch, step=NS)
        def _(rc):
            pltpu.sync_copy(zbuf, acc.at[pl.ds(rc * zrows, zrows)])

        plsc.subcore_barrier()

        @pl.loop(w, nchunk, step=NW)
        def _(ci):
            base = ci * CHUNK
            pltpu.sync_copy(row_hbm.at[pl.ds(base, CHUNK)], row_v)
            pltpu.sync_copy(col_hbm.at[pl.ds(base, CHUNK)], col_v)
            pltpu.sync_copy(ew_hbm.at[pl.ds(base, CHUNK)], ew_v)
            pltpu.sync_copy(g_hbm.at[row_v], msg)

            @pl.loop(0, CHUNK // LANES)
            def _(q):
                ew16 = ew_v[pl.ds(q * LANES, LANES)]
                for r in range(LANES):
                    bvec = _bcast_lane(ew16, r)
                    eidx = q * LANES + r
                    for k in range(d // LANES):
                        sl = pl.ds(k * LANES, LANES)
                        msg[eidx, sl] = msg[eidx, sl] * bvec

            pltpu.sync_copy(msg, acc.at[col_v], add=True)

        plsc.subcore_barrier()

        @pl.loop(s, nzch, step=NS)
        def _(rc):
            pltpu.sync_copy(acc.at[pl.ds(rc * zrows, zrows)],
                            out_hbm.at[c, pl.ds(rc * zrows, zrows)])

    return agg_kernel(g, row, col, ew)


def _dis_from_partials(d_ref):
    deg = d_ref[0, :, 0] + d_ref[1, :, 0] + 2.0
    return jnp.where(deg > 0, lax.rsqrt(jnp.where(deg > 0, deg, 1.0)), 0.0)


def _tc_transform(x, W, deg_pp):
    n, din = x.shape
    dout = W.shape[1]
    blk = 1000
    assert n % blk == 0

    def body(x_ref, w_ref, d_ref, g_ref):
        dis = _dis_from_partials(d_ref)
        h = jnp.dot(x_ref[...], w_ref[...], preferred_element_type=jnp.float32)
        g_ref[...] = dis[:, None] * h

    return pl.pallas_call(
        body,
        grid=(n // blk,),
        in_specs=[
            pl.BlockSpec((blk, din), lambda i: (i, 0)),
            pl.BlockSpec((din, dout), lambda i: (0, 0)),
            pl.BlockSpec((NC, blk, DEGW), lambda i: (0, i, 0)),
        ],
        out_specs=pl.BlockSpec((blk, dout), lambda i: (i, 0)),
        out_shape=jax.ShapeDtypeStruct((n, dout), jnp.float32),
    )(x, W, deg_pp)


def _tc_finalize(agg_pp, deg_pp, g, b):
    n, dout = g.shape
    blk = 1000
    assert n % blk == 0

    def body(a_ref, d_ref, g_ref, b_ref, o_ref):
        dis = _dis_from_partials(d_ref)
        a = a_ref[0] + a_ref[1] + 2.0 * g_ref[...]
        o_ref[...] = jnp.maximum(dis[:, None] * a + b_ref[...], 0.0)

    return pl.pallas_call(
        body,
        grid=(n // blk,),
        in_specs=[
            pl.BlockSpec((NC, blk, dout), lambda i: (0, i, 0)),
            pl.BlockSpec((NC, blk, DEGW), lambda i: (0, i, 0)),
            pl.BlockSpec((blk, dout), lambda i: (i, 0)),
            pl.BlockSpec((1, dout), lambda i: (0, 0)),
        ],
        out_specs=pl.BlockSpec((blk, dout), lambda i: (i, 0)),
        out_shape=jax.ShapeDtypeStruct((n, dout), jnp.float32),
    )(agg_pp, deg_pp, g, b.reshape(1, dout))


def kernel(x, edge_index, edge_weight, W, b):
    n = x.shape[0]
    row = edge_index[0]
    col = edge_index[1]
    deg_pp = _sc_degree(col, edge_weight, n)
    g = _tc_transform(x, W, deg_pp)
    agg_pp = _sc_aggregate(g, row, col, edge_weight, n)
    return _tc_finalize(agg_pp, deg_pp, g, b)


# speedup vs baseline: 19.2995x; 1.5133x over previous
import dataclasses
import functools

import jax
import jax.numpy as jnp
from jax import lax
from jax.experimental import pallas as pl
from jax.experimental.pallas import tpu as pltpu
from jax.experimental.pallas import tpu_sc as plsc

NC = 2
NS = 16
NW = NC * NS
LANES = 16
CHUNK = 128

_MESH = plsc.VectorSubcoreMesh(core_axis_name="c", subcore_axis_name="s")

_GATHER_DNUMS = lax.GatherDimensionNumbers(
    offset_dims=(), collapsed_slice_dims=(0,), start_index_map=(0,))


def _bcast_lane(vec16, r):
    idx = jnp.full((LANES, 1), r, jnp.int32)
    return lax.gather(vec16, idx, _GATHER_DNUMS, (1,),
                      mode=lax.GatherScatterMode.PROMISE_IN_BOUNDS)


_CP_NO_LAYOUT = pltpu.CompilerParams()
if "needs_layout_passes" in pltpu.CompilerParams.__dataclass_fields__:
    _CP_NO_LAYOUT = dataclasses.replace(_CP_NO_LAYOUT,
                                        needs_layout_passes=False)

ECH = 2000


def _sc_degree(col, ew, n):
    e = col.shape[0]
    assert e % ECH == 0
    nchunk = e // ECH
    n_pad = n
    assert n_pad % (LANES * NS) == 0
    npw = n_pad // NS

    @functools.partial(
        pl.kernel,
        out_type=(jax.ShapeDtypeStruct((n_pad,), jnp.float32),
                  jax.ShapeDtypeStruct((n_pad,), jnp.float32)),
        mesh=_MESH,
        scratch_types=[
            pltpu.VMEM((ECH,), jnp.int32),
            pltpu.VMEM((ECH,), jnp.float32),
            pltpu.VMEM((n_pad,), jnp.float32),
            pltpu.VMEM((NS, npw), jnp.float32),
            pltpu.VMEM((npw,), jnp.float32),
            pltpu.VMEM_SHARED((NS, n_pad), jnp.float32),
        ],
        compiler_params=_CP_NO_LAYOUT,
    )
    def deg_kernel(col_hbm, ew_hbm, out0_hbm, out1_hbm,
                   col_v, ew_v, hist, redbuf, res, stage):
        c = lax.axis_index("c")
        s = lax.axis_index("s")
        w = s * NC + c

        @pl.loop(0, n_pad // LANES)
        def _(j):
            hist[pl.ds(j * LANES, LANES)] = jnp.zeros((LANES,), jnp.float32)

        @pl.loop(w, nchunk, step=NW)
        def _(ci):
            base = ci * ECH
            pltpu.sync_copy(col_hbm.at[pl.ds(base, ECH)], col_v)
            pltpu.sync_copy(ew_hbm.at[pl.ds(base, ECH)], ew_v)

            @pl.loop(0, ECH // LANES)
            def _(j):
                i16 = col_v[pl.ds(j * LANES, LANES)]
                v16 = ew_v[pl.ds(j * LANES, LANES)]
                plsc.addupdate_scatter(hist, [i16], v16)

        pltpu.sync_copy(hist, stage.at[s])
        plsc.subcore_barrier()

        @pl.loop(0, NS)
        def _(t):
            pltpu.sync_copy(stage.at[t, pl.ds(s * npw, npw)], redbuf.at[t])

        @pl.loop(0, npw // LANES)
        def _(j):
            sl = pl.ds(j * LANES, LANES)
            acc16 = redbuf[0, sl]
            for t in range(1, NS):
                acc16 = acc16 + redbuf[t, sl]
            res[sl] = acc16

        @pl.when(c == 0)
        def _():
            pltpu.sync_copy(res, out0_hbm.at[pl.ds(s * npw, npw)])

        @pl.when(c == 1)
        def _():
            pltpu.sync_copy(res, out1_hbm.at[pl.ds(s * npw, npw)])

    return deg_kernel(col, ew)


def _sc_aggregate(g, row, col, ew, n):
    e = row.shape[0]
    d = g.shape[1]
    assert e % CHUNK == 0 and d % LANES == 0
    nchunk = e // CHUNK
    zrows = 80
    assert n % zrows == 0
    nzch = n // zrows

    @functools.partial(
        pl.kernel,
        out_type=jax.ShapeDtypeStruct((NC, n, d), jnp.float32),
        mesh=_MESH,
        scratch_types=[
            pltpu.VMEM((CHUNK,), jnp.int32),
            pltpu.VMEM((CHUNK,), jnp.int32),
            pltpu.VMEM((CHUNK,), jnp.float32),
            pltpu.VMEM((CHUNK, d), jnp.float32),
            pltpu.VMEM((zrows, d), jnp.float32),
            pltpu.VMEM_SHARED((n, d), jnp.float32),
        ],
    )
    def agg_kernel(g_hbm, row_hbm, col_hbm, ew_hbm, out_hbm,
                   row_v, col_v, ew_v, msg, zbuf, acc):
        c = lax.axis_index("c")
        s = lax.axis_index("s")
        w = s * NC + c

        @pl.loop(0, zrows)
        def _(r):
            @pl.loop(0, d // LANES)
            def _(k):
                zbuf[r, pl.ds(k * LANES, LANES)] = jnp.zeros((LANES,),
                                                             jnp.float32)

        @pl.loop(s, nzch, step=NS)
        def _(rc):
            pltpu.sync_copy(zbuf, acc.at[pl.ds(rc * zrows, zrows)])

        plsc.subcore_barrier()

        @pl.loop(w, nchunk, step=NW)
        def _(ci):
            base = ci * CHUNK
            pltpu.sync_copy(row_hbm.at[pl.ds(base, CHUNK)], row_v)
            pltpu.sync_copy(col_hbm.at[pl.ds(base, CHUNK)], col_v)
            pltpu.sync_copy(ew_hbm.at[pl.ds(base, CHUNK)], ew_v)
            pltpu.sync_copy(g_hbm.at[row_v], msg)

            @pl.loop(0, CHUNK // LANES)
            def _(q):
                ew16 = ew_v[pl.ds(q * LANES, LANES)]
                for r in range(LANES):
                    bvec = _bcast_lane(ew16, r)
                    eidx = q * LANES + r
                    for k in range(d // LANES):
                        sl = pl.ds(k * LANES, LANES)
                        msg[eidx, sl] = msg[eidx, sl] * bvec

            pltpu.sync_copy(msg, acc.at[col_v], add=True)

        plsc.subcore_barrier()

        @pl.loop(s, nzch, step=NS)
        def _(rc):
            pltpu.sync_copy(acc.at[pl.ds(rc * zrows, zrows)],
                            out_hbm.at[c, pl.ds(rc * zrows, zrows)])

    return agg_kernel(g, row, col, ew)


def _dis_from_partials(d0_ref, d1_ref, i, blk):
    deg = d0_ref[pl.ds(i * blk, blk)] + d1_ref[pl.ds(i * blk, blk)] + 2.0
    return jnp.where(deg > 0, lax.rsqrt(jnp.where(deg > 0, deg, 1.0)), 0.0)


def _tc_transform(x, W, deg0, deg1):
    n, din = x.shape
    dout = W.shape[1]
    n_pad = deg0.shape[0]
    blk = 1280
    assert n % blk == 0

    def body(x_ref, w_ref, d0_ref, d1_ref, g_ref):
        i = pl.program_id(0)
        dis = _dis_from_partials(d0_ref, d1_ref, i, blk)
        h = jnp.dot(x_ref[...], w_ref[...], preferred_element_type=jnp.float32)
        g_ref[...] = dis[:, None] * h

    return pl.pallas_call(
        body,
        grid=(n // blk,),
        in_specs=[
            pl.BlockSpec((blk, din), lambda i: (i, 0)),
            pl.BlockSpec((din, dout), lambda i: (0, 0)),
            pl.BlockSpec((n_pad,), lambda i: (0,)),
            pl.BlockSpec((n_pad,), lambda i: (0,)),
        ],
        out_specs=pl.BlockSpec((blk, dout), lambda i: (i, 0)),
        out_shape=jax.ShapeDtypeStruct((n, dout), jnp.float32),
    )(x, W, deg0, deg1)


def _tc_finalize(agg_pp, deg0, deg1, g, b):
    n, dout = g.shape
    n_pad = deg0.shape[0]
    blk = 1280
    assert n % blk == 0

    def body(a_ref, d0_ref, d1_ref, g_ref, b_ref, o_ref):
        i = pl.program_id(0)
        dis = _dis_from_partials(d0_ref, d1_ref, i, blk)
        a = a_ref[0] + a_ref[1] + 2.0 * g_ref[...]
        o_ref[...] = jnp.maximum(dis[:, None] * a + b_ref[...], 0.0)

    return pl.pallas_call(
        body,
        grid=(n // blk,),
        in_specs=[
            pl.BlockSpec((NC, blk, dout), lambda i: (0, i, 0)),
            pl.BlockSpec((n_pad,), lambda i: (0,)),
            pl.BlockSpec((n_pad,), lambda i: (0,)),
            pl.BlockSpec((blk, dout), lambda i: (i, 0)),
            pl.BlockSpec((1, dout), lambda i: (0, 0)),
        ],
        out_specs=pl.BlockSpec((blk, dout), lambda i: (i, 0)),
        out_shape=jax.ShapeDtypeStruct((n, dout), jnp.float32),
    )(agg_pp, deg0, deg1, g, b.reshape(1, dout))


def kernel(x, edge_index, edge_weight, W, b):
    n = x.shape[0]
    n_pad = -(-n // 1280) * 1280
    row = edge_index[0]
    col = edge_index[1]
    x_p = jnp.pad(x, ((0, n_pad - n), (0, 0)))
    deg0, deg1 = _sc_degree(col, edge_weight, n_pad)
    g = _tc_transform(x_p, W, deg0, deg1)
    agg_pp = _sc_aggregate(g, row, col, edge_weight, n_pad)
    return _tc_finalize(agg_pp, deg0, deg1, g, b)[:n]
